# baseline (speedup 1.0000x reference)
import jax
import jax.numpy as jnp
from jax.experimental import pallas as pl
from jax.experimental.pallas import tpu as pltpu
from jax.experimental.pallas import tpu_sc as plsc

_TB = 256
_SC_ROWS = 8
_SC_LANES = 16


def _tc_add_kernel(x_ref, pos_ref, out_ref):
    out_ref[...] = x_ref[...] + pos_ref[...][None, :, :]


def _tc_part(x, pos_table, nb):
    B, T, D = x.shape
    return pl.pallas_call(
        _tc_add_kernel,
        grid=(T // _TB,),
        in_specs=[
            pl.BlockSpec((nb, _TB, D), lambda i: (0, i, 0)),
            pl.BlockSpec((_TB, D), lambda i: (i, 0)),
        ],
        out_specs=pl.BlockSpec((nb, _TB, D), lambda i: (0, i, 0)),
        out_shape=jax.ShapeDtypeStruct((nb, T, D), x.dtype),
    )(x, pos_table)


def _sc_full(x, pos_table):
    B, T, D = x.shape
    mesh = plsc.VectorSubcoreMesh(core_axis_name="core", subcore_axis_name="subcore")

    @pl.kernel(out_type=jax.ShapeDtypeStruct((B, T, D), x.dtype), mesh=mesh)
    def sc_kernel(x_hbm, pos_hbm, o_hbm):
        def body(x_vmem, pos_vmem, o_vmem):
            x2, o2 = x_vmem.at[0], o_vmem.at[0]

            @pl.loop(0, _SC_ROWS)
            def _(r):
                @pl.loop(0, D, step=_SC_LANES)
                def _(c):
                    slc = (pl.ds(r, 1), pl.ds(c, _SC_LANES))
                    o2.at[*slc][...] = x2.at[*slc][...] + pos_vmem.at[*slc][...]

        pltpu.emit_pipeline(
            body,
            grid=(B, T // _SC_ROWS),
            in_specs=[
                pl.BlockSpec((1, _SC_ROWS, D), index_map=lambda b, i: (b, i, 0)),
                pl.BlockSpec((_SC_ROWS, D), index_map=lambda b, i: (i, 0)),
            ],
            out_specs=[
                pl.BlockSpec((1, _SC_ROWS, D), index_map=lambda b, i: (b, i, 0))
            ],
            core_axis_name=("core", "subcore"),
            dimension_semantics=(pltpu.PARALLEL, pltpu.PARALLEL),
        )(x_hbm, pos_hbm, o_hbm)

    return sc_kernel(x, pos_table)


def _tc_part_last(x, pos_table):
    B, T, D = x.shape
    return pl.pallas_call(
        _tc_add_kernel,
        grid=(T // _TB,),
        in_specs=[
            pl.BlockSpec((1, _TB, D), lambda i: (B - 1, i, 0)),
            pl.BlockSpec((_TB, D), lambda i: (i, 0)),
        ],
        out_specs=pl.BlockSpec((1, _TB, D), lambda i: (0, i, 0)),
        out_shape=jax.ShapeDtypeStruct((1, T, D), x.dtype),
    )(x, pos_table)


def kernel(x, pos_table):
    return _sc_full(x, pos_table)


# speedup vs baseline: 3.8328x; 3.8328x over previous
import jax
import jax.numpy as jnp
from jax.experimental import pallas as pl

_TB = 1024


def _add_kernel(x_ref, pos_ref, out_ref):
    out_ref[...] = x_ref[...] + pos_ref[...][None, :, :]


def kernel(x, pos_table):
    B, T, D = x.shape
    grid = (T // _TB, B)
    return pl.pallas_call(
        _add_kernel,
        grid=grid,
        in_specs=[
            pl.BlockSpec((1, _TB, D), lambda i, b: (b, i, 0)),
            pl.BlockSpec((_TB, D), lambda i, b: (i, 0)),
        ],
        out_specs=pl.BlockSpec((1, _TB, D), lambda i, b: (b, i, 0)),
        out_shape=jax.ShapeDtypeStruct((B, T, D), x.dtype),
    )(x, pos_table)
